# baseline (speedup 1.0000x reference)
import jax
import jax.numpy as jnp
from jax import lax
from jax.experimental import pallas as pl
from jax.experimental.pallas import tpu as pltpu
from jax.experimental.pallas import tpu_sc as plsc

N = 1048576
B = 32768
NELEM = 119
L = 16
NC, NS = 2, 16
NW = NC * NS
CH = N // NW
NWIN = 4
WCH = CH // NWIN
WNV = WCH // L


def _sc_body(feat_hbm, batch_hbm, table_hbm, out_hbm,
             feat_v, batch_v, table_v, acc_v,
             tsem, fsems, bsems):
    wid = lax.axis_index("s") * NC + lax.axis_index("c")
    base = wid * CH

    ct = pltpu.async_copy(table_hbm, table_v, tsem)
    copies = []
    for w in range(NWIN):
        o = w * WCH
        cf = pltpu.async_copy(feat_hbm.at[pl.ds(base + o, WCH)],
                              feat_v.at[pl.ds(o, WCH)], fsems[w])
        cb = pltpu.async_copy(batch_hbm.at[pl.ds(base + o, WCH)],
                              batch_v.at[pl.ds(o, WCH)], bsems[w])
        copies.append((cf, cb))

    zero = jnp.zeros((L,), jnp.float32)

    @plsc.parallel_loop(0, B // L, unroll=8)
    def _zero(i):
        acc_v[pl.ds(i * L, L)] = zero

    ct.wait()

    lane = lax.iota(jnp.int32, L)
    is_last = lane == (L - 1)
    not_last = lane < (L - 1)

    for w in range(NWIN):
        copies[w][0].wait()
        copies[w][1].wait()

        @plsc.parallel_loop(w * WNV, (w + 1) * WNV, unroll=4)
        def _main(i):
            off = i * L
            f = feat_v[pl.ds(off, L)]
            s = batch_v[pl.ds(off, L)]
            sn = batch_v[pl.ds(off + 1, L)]
            vals = plsc.load_gather(table_v, [f])
            t = plsc.cumsum(vals)
            mb = s != sn
            m_plus = jnp.logical_or(mb, is_last)
            m_minus = jnp.logical_and(mb, not_last)
            plsc.addupdate_scatter(acc_v, [s], t, mask=m_plus)
            plsc.addupdate_scatter(acc_v, [sn], -t, mask=m_minus)

    pltpu.sync_copy(acc_v, out_hbm.at[wid])


def _tc_reduce(p_ref, o_ref):
    o_ref[...] = jnp.sum(p_ref[...], axis=0, keepdims=True)


def kernel(atom_features, batch_atom, property_per_element):
    feat = atom_features.reshape(N)

    mesh = plsc.VectorSubcoreMesh(core_axis_name="c", subcore_axis_name="s")
    partials = pl.kernel(
        _sc_body,
        out_type=jax.ShapeDtypeStruct((NW, B), jnp.float32),
        mesh=mesh,
        scratch_types=[
            pltpu.VMEM((CH,), jnp.int32),
            pltpu.VMEM((CH + L,), jnp.int32),
            pltpu.VMEM((NELEM,), jnp.float32),
            pltpu.VMEM((B,), jnp.float32),
            pltpu.SemaphoreType.DMA,
            [pltpu.SemaphoreType.DMA] * NWIN,
            [pltpu.SemaphoreType.DMA] * NWIN,
        ],
        compiler_params=pltpu.CompilerParams(
            needs_layout_passes=False,
            disable_bounds_checks=True,
            disable_semaphore_checks=True,
        ),
    )(feat, batch_atom, property_per_element)

    blk = B // 8
    out = pl.pallas_call(
        _tc_reduce,
        grid=(8,),
        in_specs=[pl.BlockSpec((NW, blk), lambda i: (0, i))],
        out_specs=pl.BlockSpec((1, blk), lambda i: (0, i)),
        out_shape=jax.ShapeDtypeStruct((1, B), jnp.float32),
    )(partials)
    return out.reshape(B, 1)


# speedup vs baseline: 1.0800x; 1.0800x over previous
import jax
import jax.numpy as jnp
from jax import lax
from jax.experimental import pallas as pl
from jax.experimental.pallas import tpu as pltpu
from jax.experimental.pallas import tpu_sc as plsc

N = 1048576
B = 32768
NELEM = 119
L = 16
NC, NS = 2, 16
NW = NC * NS
CH = N // NW
NWIN = 4
WCH = CH // NWIN
WNV = WCH // L


def _sc_body(feat_hbm, batch_hbm, table_hbm, out_hbm,
             feat_v, batch_v, table_v, acc_v,
             tsem, fsems, bsems):
    wid = lax.axis_index("s") * NC + lax.axis_index("c")
    base = wid * CH

    ct = pltpu.async_copy(table_hbm, table_v, tsem)
    copies = []
    for w in range(NWIN):
        o = w * WCH
        cf = pltpu.async_copy(feat_hbm.at[pl.ds(base + o, WCH)],
                              feat_v.at[pl.ds(o, WCH)], fsems[w])
        cb = pltpu.async_copy(batch_hbm.at[pl.ds(base + o, WCH)],
                              batch_v.at[pl.ds(o, WCH)], bsems[w])
        copies.append((cf, cb))

    zero = jnp.zeros((L,), jnp.float32)

    @plsc.parallel_loop(0, B // L, unroll=8)
    def _zero(i):
        acc_v[pl.ds(i * L, L)] = zero

    ct.wait()

    lane = lax.iota(jnp.int32, L)
    is_last = lane == (L - 1)
    not_last = lane < (L - 1)

    for w in range(NWIN):
        copies[w][0].wait()
        copies[w][1].wait()

        @plsc.parallel_loop(w * WNV, (w + 1) * WNV, unroll=4)
        def _main(i):
            off = i * L
            f = feat_v[pl.ds(off, L)]
            s = batch_v[pl.ds(off, L)]
            sn = batch_v[pl.ds(off + 1, L)]
            vals = plsc.load_gather(table_v, [f])
            t = plsc.cumsum(vals)
            mb = s != sn
            m_plus = jnp.logical_or(mb, is_last)
            m_minus = jnp.logical_and(mb, not_last)
            plsc.addupdate_scatter(acc_v, [s], t, mask=m_plus)
            plsc.addupdate_scatter(acc_v, [sn], -t, mask=m_minus)

    pltpu.sync_copy(acc_v, out_hbm.at[wid])


def _tc_reduce(p_ref, o_ref):
    o_ref[...] = jnp.sum(p_ref[...], axis=0, keepdims=True)


def kernel(atom_features, batch_atom, property_per_element):
    feat = atom_features.reshape(N)

    mesh = plsc.VectorSubcoreMesh(core_axis_name="c", subcore_axis_name="s")
    partials = pl.kernel(
        _sc_body,
        out_type=jax.ShapeDtypeStruct((NW, B), jnp.float32),
        mesh=mesh,
        scratch_types=[
            pltpu.VMEM((CH,), jnp.int32),
            pltpu.VMEM((CH + L,), jnp.int32),
            pltpu.VMEM((NELEM,), jnp.float32),
            pltpu.VMEM((B,), jnp.float32),
            pltpu.SemaphoreType.DMA,
            [pltpu.SemaphoreType.DMA] * NWIN,
            [pltpu.SemaphoreType.DMA] * NWIN,
        ],
        compiler_params=pltpu.CompilerParams(
            needs_layout_passes=False,
            disable_bounds_checks=True,
            disable_semaphore_checks=True,
        ),
    )(feat, batch_atom, property_per_element)

    out = pl.pallas_call(
        _tc_reduce,
        out_shape=jax.ShapeDtypeStruct((1, B), jnp.float32),
    )(partials)
    return out.reshape(B, 1)
